# Initial kernel scaffold; baseline (speedup 1.0000x reference)
import jax
import jax.numpy as jnp
from jax.experimental import pallas as pl


def kernel(x, Wg, bg, W1, b1, W3, b3, W2, b2):
    raise NotImplementedError("write your pallas kernel here")


# speedup vs baseline: 1.0375x; 1.0375x over previous
import functools

import jax
import jax.numpy as jnp
from jax import lax
from jax.experimental import pallas as pl
from jax.experimental.pallas import tpu as pltpu
from jax.experimental.pallas import tpu_sc as plsc

T, D, I, E, K = 2048, 768, 1024, 8, 2
RB = 128
NSLOT = 5120
NBLK = NSLOT // RB
JB = 512
NC, NS = 2, 16
NW = NC * NS
NEG = -1e30


def _fiota(shape, dim):
    return lax.broadcasted_iota(jnp.int32, shape, dim).astype(jnp.float32)


def _route_body(x_ref, wg_ref, bg_ref, pos0_ref, pos1_ref, st_ref,
                wslot_ref, be_ref):
    xb = x_ref[...]
    logits = jnp.dot(xb, wg_ref[...], preferred_element_type=jnp.float32)
    logits = logits + bg_ref[...]
    m = jnp.max(logits, axis=-1, keepdims=True)
    p = jnp.exp(logits - m)
    s = p / jnp.sum(p, axis=-1, keepdims=True)

    ei = _fiota((T, E), 1)
    m0 = jnp.max(s, axis=-1, keepdims=True)
    i0 = jnp.min(jnp.where(s >= m0, ei, float(E)), axis=-1, keepdims=True)
    s2 = jnp.where(ei == i0, NEG, s)
    m1 = jnp.max(s2, axis=-1, keepdims=True)
    i1 = jnp.min(jnp.where(s2 >= m1, ei, float(E)), axis=-1, keepdims=True)

    m0e = (ei == i0).astype(jnp.float32)
    m1e = (ei == i1).astype(jnp.float32)

    ri = _fiota((T, T), 0)
    ci = _fiota((T, T), 1)
    ltri = (ci <= ri).astype(jnp.float32)
    c01 = jnp.dot(ltri, jnp.concatenate([m0e, m1e], axis=1),
                  preferred_element_type=jnp.float32)
    c0 = c01[:, :E]
    c1 = c01[:, E:]

    n0 = c0[T - 1:T, :]
    n1 = c1[T - 1:T, :]
    n = n0 + n1
    nblk = jnp.floor((n + (RB - 1.0)) * (1.0 / RB))
    r8 = _fiota((E, E), 0)
    c8 = _fiota((E, E), 1)
    sutri = (r8 < c8).astype(jnp.float32)
    blkoff = jnp.dot(nblk, sutri, preferred_element_type=jnp.float32)
    off = blkoff * RB

    pos0 = jnp.sum(m0e * (off + c0 - m0e), axis=-1, keepdims=True)
    pos1 = jnp.sum(m1e * (off + n0 + c1 - m1e), axis=-1, keepdims=True)
    pos0_ref[...] = pos0.astype(jnp.int32)
    pos1_ref[...] = pos1.astype(jnp.int32)

    bi = _fiota((E, NBLK), 1)
    eb = _fiota((E, NBLK), 0)
    boffc = jnp.reshape(blkoff, (E, 1))
    nblkc = jnp.reshape(nblk, (E, 1))
    ind = ((bi >= boffc) & (bi < boffc + nblkc)).astype(jnp.float32)
    bex = jnp.sum(eb * ind, axis=0, keepdims=True)
    bex = bex + (E - 1.0) * (1.0 - jnp.sum(ind, axis=0, keepdims=True))
    be_ref[...] = bex.astype(jnp.int32)

    tok = _fiota((T, 1), 0)
    for jb in range(NSLOT // JB):
        jcol = _fiota((1, JB), 1) + (jb * JB)
        o0 = (pos0 == jcol).astype(jnp.float32)
        o1 = (pos1 == jcol).astype(jnp.float32)
        dn = (((0,), (0,)), ((), ()))
        st_b = lax.dot_general(o0 + o1, tok, dn,
                               preferred_element_type=jnp.float32)
        ws_b = (lax.dot_general(o0, m0, dn, preferred_element_type=jnp.float32)
                + lax.dot_general(o1, m1, dn,
                                  preferred_element_type=jnp.float32))
        st_ref[pl.ds(jb * JB, JB), :] = st_b.astype(jnp.int32)
        wslot_ref[pl.ds(jb * JB, JB), :] = ws_b


def _route(x, Wg, bg):
    return pl.pallas_call(
        _route_body,
        out_shape=(
            jax.ShapeDtypeStruct((T, 1), jnp.int32),
            jax.ShapeDtypeStruct((T, 1), jnp.int32),
            jax.ShapeDtypeStruct((NSLOT, 1), jnp.int32),
            jax.ShapeDtypeStruct((NSLOT, 1), jnp.float32),
            jax.ShapeDtypeStruct((1, NBLK), jnp.int32),
        ),
    )(x, Wg, bg)


_BPW = NSLOT // NW


@functools.cache
def _sc_mesh():
    return plsc.VectorSubcoreMesh(core_axis_name="c", subcore_axis_name="s",
                                  num_cores=NC, num_subcores=NS)


@functools.cache
def _make_dispatch():
    @functools.partial(
        pl.kernel,
        out_type=jax.ShapeDtypeStruct((NSLOT, D), jnp.float32),
        mesh=_sc_mesh(),
        scratch_types=[
            pltpu.VMEM((_BPW,), jnp.int32),
            pltpu.VMEM((_BPW, D), jnp.float32),
            pltpu.SemaphoreType.DMA,
        ],
    )
    def _dispatch(x_hbm, st_hbm, xs_hbm, idx_v, rows_v, sem):
        wid = lax.axis_index("s") * NC + lax.axis_index("c")
        base = wid * _BPW
        pltpu.sync_copy(st_hbm.at[pl.ds(base, _BPW)], idx_v)
        pltpu.async_copy(x_hbm.at[idx_v], rows_v, sem).wait()
        pltpu.sync_copy(rows_v, xs_hbm.at[pl.ds(base, _BPW)])

    return _dispatch


def _ffn_body(be_ref, xs_ref, w1_ref, b1_ref, w3_ref, b3_ref, w2_ref, b2_ref,
              ws_ref, out_ref):
    xsb = xs_ref[...]
    g = jnp.dot(xsb, w1_ref[0], preferred_element_type=jnp.float32) + b1_ref[0]
    u = jnp.dot(xsb, w3_ref[0], preferred_element_type=jnp.float32) + b3_ref[0]
    h = g * (1.0 / (1.0 + jnp.exp(-g))) * u
    o = jnp.dot(h, w2_ref[0], preferred_element_type=jnp.float32) + b2_ref[0]
    out_ref[...] = o * ws_ref[0]


def _ffn(be, xs, W1, b1, W3, b3, W2, b2, wslot):
    grid_spec = pltpu.PrefetchScalarGridSpec(
        num_scalar_prefetch=1,
        grid=(NBLK,),
        in_specs=[
            pl.BlockSpec((RB, D), lambda b, be_ref: (b, 0)),
            pl.BlockSpec((1, D, I), lambda b, be_ref: (be_ref[b], 0, 0)),
            pl.BlockSpec((1, 1, I), lambda b, be_ref: (be_ref[b], 0, 0)),
            pl.BlockSpec((1, D, I), lambda b, be_ref: (be_ref[b], 0, 0)),
            pl.BlockSpec((1, 1, I), lambda b, be_ref: (be_ref[b], 0, 0)),
            pl.BlockSpec((1, I, D), lambda b, be_ref: (be_ref[b], 0, 0)),
            pl.BlockSpec((1, 1, D), lambda b, be_ref: (be_ref[b], 0, 0)),
            pl.BlockSpec((1, RB, 1), lambda b, be_ref: (b, 0, 0)),
        ],
        out_specs=pl.BlockSpec((RB, D), lambda b, be_ref: (b, 0)),
    )
    return pl.pallas_call(
        _ffn_body,
        grid_spec=grid_spec,
        out_shape=jax.ShapeDtypeStruct((NSLOT, D), jnp.float32),
        compiler_params=pltpu.CompilerParams(
            dimension_semantics=("arbitrary",),
        ),
    )(be, xs, W1, b1, W3, b3, W2, b2, wslot)


_TPW = T // NW


@functools.cache
def _make_combine():
    @functools.partial(
        pl.kernel,
        out_type=jax.ShapeDtypeStruct((T, D), jnp.float32),
        mesh=_sc_mesh(),
        scratch_types=[
            pltpu.VMEM((_TPW,), jnp.int32),
            pltpu.VMEM((_TPW,), jnp.int32),
            pltpu.VMEM((_TPW, D), jnp.float32),
            pltpu.VMEM((_TPW, D), jnp.float32),
            pltpu.SemaphoreType.DMA,
            pltpu.SemaphoreType.DMA,
        ],
    )
    def _combine(out2_hbm, pos0_hbm, pos1_hbm, y_hbm, i0_v, i1_v, r0_v, r1_v,
                 sem0, sem1):
        wid = lax.axis_index("s") * NC + lax.axis_index("c")
        base = wid * _TPW
        pltpu.sync_copy(pos0_hbm.at[pl.ds(base, _TPW)], i0_v)
        pltpu.sync_copy(pos1_hbm.at[pl.ds(base, _TPW)], i1_v)
        cp0 = pltpu.async_copy(out2_hbm.at[i0_v], r0_v, sem0)
        cp1 = pltpu.async_copy(out2_hbm.at[i1_v], r1_v, sem1)
        cp0.wait()
        cp1.wait()

        def row(r, _):
            def col(c, __):
                sl = pl.ds(c * 16, 16)
                r0_v[r, sl] = r0_v[r, sl] + r1_v[r, sl]
                return __
            return lax.fori_loop(0, D // 16, col, _)

        lax.fori_loop(0, _TPW, row, 0)
        pltpu.sync_copy(r0_v, y_hbm.at[pl.ds(base, _TPW)])

    return _combine


def kernel(x, Wg, bg, W1, b1, W3, b3, W2, b2):
    pos0, pos1, st, wslot, be = _route(x, Wg, jnp.reshape(bg, (1, E)))
    xs = _make_dispatch()(x, jnp.reshape(st, (NSLOT,)))
    out2 = _ffn(jnp.reshape(be, (NBLK,)), xs,
                W1, jnp.reshape(b1, (E, 1, I)),
                W3, jnp.reshape(b3, (E, 1, I)),
                W2, jnp.reshape(b2, (E, 1, D)),
                jnp.reshape(wslot, (NBLK, RB, 1)))
    y = _make_combine()(out2, jnp.reshape(pos0, (T,)), jnp.reshape(pos1, (T,)))
    return y
